# baseline (speedup 1.0000x reference)
import functools
import jax
import jax.numpy as jnp
import numpy as np
from jax import lax
from jax.experimental import pallas as pl
from jax.experimental.pallas import tpu as pltpu
from jax.experimental.pallas import tpu_sc as plsc

N = 2; T = 16; H = 128; W = 128; CF = 64
NP_ = 100; NC_ = 133; IGNORE = 255
RANK_W = 0.1; SEG_W = 1.0; MASK_W = 1.0; DICE_W = 4.0; CLS_W = 2.0; INST_W = 1.0
KS = 4096; ST = 0.3; MC = -99999.0
HW = H * W
JT = 512
NJ = KS // JT

def _gumbel_const():
    return -jnp.log(-jnp.log(jax.random.uniform(
        jax.random.key(42), (N, HW), minval=1e-6, maxval=1.0 - 1e-6)))

try:
    _GUMBEL = np.asarray(_gumbel_const())
except Exception:
    _GUMBEL = None


def _softplus(x):
    return jnp.maximum(x, 0.0) + jnp.log1p(jnp.exp(-jnp.abs(x)))


def _match_body(pm_ref, tm_ref, plg_ref, lab_ref, gum_ref, misc_ref, key_ref, meta_ref):
    x = pm_ref[0]
    t = tm_ref[0]
    p = jnp.clip(jax.nn.sigmoid(x), 1e-6, 1.0 - 1e-6)
    dn = (((1,), (1,)), ((), ()))
    pt = lax.dot_general(p, t, dn, preferred_element_type=jnp.float32)
    ones_hw = jnp.ones((1, HW), jnp.float32)
    t_area = lax.dot_general(ones_hw, t, dn, preferred_element_type=jnp.float32)
    p_sum = jnp.sum(p, axis=1, keepdims=True)
    mask_cost = (t_area + p_sum - 2.0 * pt) / HW
    dice_cost = -(2.0 * pt) / (p_sum + t_area + 1e-6)
    xl = plg_ref[0]
    prob = jax.nn.sigmoid(xl)
    neg = 0.75 * prob * prob * (-jnp.log(1.0 - prob + 1e-8))
    pos = 0.25 * (1.0 - prob) * (1.0 - prob) * (-jnp.log(prob + 1e-8))
    pn = pos - neg
    lab = lab_ref[0]
    ciota = lax.broadcasted_iota(jnp.int32, (NC_, T), 0)
    oh = jnp.where(ciota == lab, 1.0, 0.0)
    clsc = lax.dot_general(pn, oh, (((1,), (0,)), ((), ())),
                           preferred_element_type=jnp.float32)
    cost0 = MASK_W * mask_cost + DICE_W * dice_cost + CLS_W * clsc

    riota = lax.broadcasted_iota(jnp.int32, (NP_, T), 0)
    cio = lax.broadcasted_iota(jnp.int32, (NP_, T), 1)
    flat = riota * T + cio
    lane = lax.broadcasted_iota(jnp.int32, (1, 128), 1)

    def step(s, carry):
        cost, misc = carry
        mn = jnp.min(cost)
        fi = jnp.min(jnp.where(cost == mn, flat, 10 ** 9))
        i = fi // T
        j = fi - i * T
        cost = jnp.where((riota == i) | (cio == j), jnp.inf, cost)
        misc = jnp.where(lane == s, i.astype(jnp.float32), misc)
        misc = jnp.where(lane == T + s, j.astype(jnp.float32), misc)
        return cost, misc

    _, misc = lax.fori_loop(0, T, step, (cost0, jnp.zeros((1, 128), jnp.float32)))

    term0 = jnp.sum(0.75 * prob * prob * _softplus(xl))
    misc = misc + jnp.where(lane == 2 * T, term0, 0.0)

    tmr = jnp.round(t)
    area = lax.dot_general(ones_hw, tmr, dn, preferred_element_type=jnp.float32)
    dn2 = (((1,), (0,)), ((), ()))
    pix = lax.dot_general(area, tmr, dn2, preferred_element_type=jnp.float32)
    pix = jnp.where(pix == 0.0, 1.0, pix)
    cover = lax.dot_general(jnp.ones((1, T), jnp.float32), tmr, dn2,
                            preferred_element_type=jnp.float32)
    nonvoid = jnp.where(cover == 0.0, 0.0, 1.0)
    z = jnp.log(HW / pix) * ST + (1.0 - nonvoid) * MC + gum_ref[0]
    bits = lax.bitcast_convert_type(z, jnp.int32)
    keys = bits ^ lax.shift_right_arithmetic(bits, 31).astype(jnp.int32).__and__(
        jnp.int32(0x7FFFFFFF))
    key_ref[0] = keys
    cnt0 = jnp.sum(jnp.where(keys >= 0, 1.0, 0.0))
    lo0 = jnp.where(cnt0 >= KS, jnp.int32(0), jnp.int32(-2147483648))
    hi0 = jnp.where(cnt0 >= KS, jnp.int32(2147483647), jnp.int32(-1))

    def bstep(_, carry):
        lo, hi = carry
        d = hi - lo
        mid = lo + lax.shift_right_logical(d, 1) + (d & 1)
        cnt = jnp.sum(jnp.where(keys >= mid, 1.0, 0.0))
        ok = cnt >= KS
        return jnp.where(ok, mid, lo), jnp.where(ok, hi, mid - 1)

    thr, _ = lax.fori_loop(0, 31, bstep, (lo0, hi0))
    g = jnp.sum(jnp.where(keys > thr, 1.0, 0.0)).astype(jnp.int32)
    needed = KS - g
    r3 = lax.broadcasted_iota(jnp.int32, (3, 16), 0)
    meta_ref[0] = jnp.where(r3 == 0, thr, jnp.where(r3 == 1, g, needed))
    misc_ref[0] = misc


def _k1(pm, tm, plg, lab, gum):
    return pl.pallas_call(
        _match_body,
        grid=(N,),
        in_specs=[
            pl.BlockSpec((1, NP_, HW), lambda b: (b, 0, 0)),
            pl.BlockSpec((1, T, HW), lambda b: (b, 0, 0)),
            pl.BlockSpec((1, NP_, NC_), lambda b: (b, 0, 0)),
            pl.BlockSpec((1, 1, T), lambda b: (b, 0, 0)),
            pl.BlockSpec((1, 1, HW), lambda b: (b, 0, 0)),
        ],
        out_specs=[
            pl.BlockSpec((1, 1, 128), lambda b: (b, 0, 0)),
            pl.BlockSpec((1, 1, HW), lambda b: (b, 0, 0)),
            pl.BlockSpec((1, 3, 16), lambda b: (b, 0, 0)),
        ],
        out_shape=[
            jax.ShapeDtypeStruct((N, 1, 128), jnp.float32),
            jax.ShapeDtypeStruct((N, 1, HW), jnp.int32),
            jax.ShapeDtypeStruct((N, 3, 16), jnp.int32),
        ],
    )(pm, tm, plg, lab, gum)


CHUNK = HW // 16
NV = CHUNK // 16
OUT_LEN = KS + HW


def _pos_body(key_ref, meta_ref, pos_ref):
    x = key_ref[0]
    mrow = meta_ref[0]
    thr = jnp.max(mrow[0:1, :])
    g = jnp.max(mrow[1:2, :]).astype(jnp.float32)
    needed = jnp.max(mrow[2:3, :]).astype(jnp.float32)
    gt = x > thr
    eq = x == thr
    ri = lax.broadcasted_iota(jnp.int32, (128, 128), 0)
    ci = lax.broadcasted_iota(jnp.int32, (128, 128), 1)
    ut = jnp.where(ri <= ci, 1.0, 0.0)
    ls = jnp.where(ci < ri, 1.0, 0.0)
    dn = (((1,), (0,)), ((), ()))
    gtf = jnp.where(gt, 1.0, 0.0)
    eqf = jnp.where(eq, 1.0, 0.0)
    gp = lax.dot_general(gtf, ut, dn, preferred_element_type=jnp.float32)
    go = lax.dot_general(ls, gp[:, 127:128], dn, preferred_element_type=jnp.float32)
    gpos = gp + go - 1.0
    ep = lax.dot_general(eqf, ut, dn, preferred_element_type=jnp.float32)
    eo = lax.dot_general(ls, ep[:, 127:128], dn, preferred_element_type=jnp.float32)
    epos = ep + eo - 1.0
    sel_eq = eq & (epos < needed)
    pixf = (ri * 128 + ci).astype(jnp.float32)
    posf = jnp.where(gt, gpos, jnp.where(sel_eq, g + epos, float(KS) + pixf))
    pos_ref[0] = posf.astype(jnp.int32).reshape(16, 8, 128)


def _k1b(keys3, meta):
    return pl.pallas_call(
        _pos_body,
        grid=(N,),
        in_specs=[
            pl.BlockSpec((1, 128, 128), lambda b: (b, 0, 0)),
            pl.BlockSpec((1, 3, 16), lambda b: (b, 0, 0)),
        ],
        out_specs=pl.BlockSpec((1, 16, 8, 128), lambda b: (b, 0, 0, 0)),
        out_shape=jax.ShapeDtypeStruct((N, 16, 8, 128), jnp.int32),
    )(keys3, meta)


def _sc_sel_body(pos_hbm, packed_hbm, out_hbm, posb, buf0, buf1, sem):
    b = lax.axis_index("c")
    chunk = lax.axis_index("s")
    base = chunk * CHUNK
    pltpu.sync_copy(pos_hbm.at[b, chunk], posb)
    bufs = [buf0, buf1]
    descs = []
    for s in range(8):
        buf = bufs[s % 2]
        if s >= 2:
            descs[s - 2].wait()
        pltpu.sync_copy(packed_hbm.at[b, pl.ds(base + s * 128, 128)], buf)
        descs.append(pltpu.async_copy(buf, out_hbm.at[b].at[posb.at[s]], sem))
    descs[6].wait()
    descs[7].wait()


def _sc_select(pos4, packed):
    mesh = plsc.VectorSubcoreMesh(core_axis_name="c", subcore_axis_name="s")
    fn = functools.partial(
        pl.kernel,
        mesh=mesh,
        out_type=jax.ShapeDtypeStruct((N, OUT_LEN, 128), jnp.float32),
        scratch_types=[
            pltpu.VMEM((8, 128), jnp.int32),
            pltpu.VMEM((128, 128), jnp.float32),
            pltpu.VMEM((128, 128), jnp.float32),
            pltpu.SemaphoreType.DMA,
        ],
    )(_sc_sel_body)
    return fn(pos4, packed)


SEG_TILE = 4096
NSEG = HW // SEG_TILE


def _seg_body(sp_ref, ss_ref, out_ref):
    j = pl.program_id(1)
    x = sp_ref[0]
    m = jnp.max(x, axis=0, keepdims=True)
    lse = m + jnp.log(jnp.sum(jnp.exp(x - m), axis=0, keepdims=True))
    idx = ss_ref[0, 0]
    valid = (idx >= 0) & (idx < NC_) & (idx != IGNORE)
    vf = valid.astype(jnp.float32)
    idxc = jnp.clip(idx, 0, NC_ - 1)
    rio = lax.broadcasted_iota(jnp.int32, (NC_, SEG_TILE), 0)
    xg = jnp.sum(jnp.where(rio == idxc, x, 0.0), axis=0, keepdims=True)
    s1 = jnp.sum((lse - xg) * vf)
    s2 = jnp.sum(vf)
    lane = lax.broadcasted_iota(jnp.int32, (1, 128), 1)
    contrib = jnp.where(lane == 0, s1, 0.0) + jnp.where(lane == 1, s2, 0.0)

    @pl.when(j == 0)
    def _():
        out_ref[0] = contrib

    @pl.when(j > 0)
    def _():
        out_ref[0] = out_ref[0] + contrib


def _k2(smp, ss3):
    return pl.pallas_call(
        _seg_body,
        grid=(N, NSEG),
        in_specs=[
            pl.BlockSpec((1, NC_, SEG_TILE), lambda b, j: (b, 0, j)),
            pl.BlockSpec((1, 1, 1, SEG_TILE), lambda b, j: (b, j, 0, 0)),
        ],
        out_specs=pl.BlockSpec((1, 1, 128), lambda b, j: (b, 0, 0)),
        out_shape=jax.ShapeDtypeStruct((N, 1, 128), jnp.float32),
    )(smp, ss3)


def _inst_body(sel_ref, out_ref):
    x = sel_ref[0]
    f = x[:, :CF]
    nrm = jnp.sqrt(jnp.sum(f * f, axis=1, keepdims=True))
    fn = f / jnp.maximum(nrm, 1e-12)
    a = jnp.round(x[:, CF:CF + T])
    cnt = jnp.sum(a, axis=0, keepdims=True)
    dn_l = (((1,), (1,)), ((), ()))
    nc = lax.dot_general(a, cnt, dn_l, preferred_element_type=jnp.float32)
    ncw = jnp.where(nc == 0.0, 1.0, nc)
    w = a / ncw
    wi = jnp.sum(w, axis=0, keepdims=True)
    v = lax.dot_general(w, fn, (((0,), (0,)), ((), ())),
                        preferred_element_type=jnp.float32)
    q = lax.dot_general(wi, a, dn_l, preferred_element_type=jnp.float32)
    av = lax.dot_general(a, v, (((1,), (0,)), ((), ())),
                         preferred_element_type=jnp.float32)
    r_sum = jnp.sum(av * fn) / ST

    ones_row = jnp.ones((1, JT), jnp.float32)
    cs = [jnp.zeros((1, JT), jnp.float32) for _ in range(NJ)]
    tiles = [fn[ti * JT:(ti + 1) * JT, :].astype(jnp.bfloat16) for ti in range(NJ)]
    for ti in range(NJ):
        for tj in range(ti, NJ):
            s = lax.dot_general(tiles[ti], tiles[tj], dn_l,
                                preferred_element_type=jnp.float32) * (1.0 / ST)
            e = jnp.exp(s)
            cs[tj] = cs[tj] + lax.dot_general(ones_row, e, (((1,), (0,)), ((), ())),
                                              preferred_element_type=jnp.float32)
            if tj > ti:
                cs[ti] = cs[ti] + lax.dot_general(ones_row, e, dn_l,
                                                  preferred_element_type=jnp.float32)
    cq = jnp.zeros((), jnp.float32)
    for tj in range(NJ):
        qc = q[:, tj * JT:(tj + 1) * JT]
        cq = cq + jnp.sum(jnp.log(cs[tj]) * qc)
    lane = lax.broadcasted_iota(jnp.int32, (1, 128), 1)
    out_ref[0] = jnp.where(lane == 0, cq, 0.0) + jnp.where(lane == 1, r_sum, 0.0)


def _k34(sel):
    return pl.pallas_call(
        _inst_body,
        grid=(N,),
        in_specs=[
            pl.BlockSpec((1, KS, 128), lambda b: (b, 0, 0)),
        ],
        out_specs=pl.BlockSpec((1, 1, 128), lambda b: (b, 0, 0)),
        out_shape=jax.ShapeDtypeStruct((N, 1, 128), jnp.float32),
    )(sel)


def _post_body(si_ref, ti_ref, lab_ref, pm_ref, tm_ref, plg_ref, st_ref, rk_ref):
    b = pl.program_id(0)
    t = pl.program_id(1)
    x = pm_ref[0, 0]
    pos = tm_ref[0, 0]
    bce = jnp.sum(jnp.maximum(x, 0.0) - x * pos + jnp.log1p(jnp.exp(-jnp.abs(x))))
    sig = jax.nn.sigmoid(x)
    num = jnp.sum(sig * pos)
    dsp = jnp.sum(sig)
    dst = jnp.sum(pos)
    row = plg_ref[0, 0]
    p1 = jax.nn.sigmoid(row)
    delta = 0.25 * (1.0 - p1) * (1.0 - p1) * _softplus(-row) \
        - 0.75 * p1 * p1 * _softplus(row)
    ti = ti_ref[b, t]
    labv = lab_ref[b, ti]
    cio = lax.broadcasted_iota(jnp.int32, (1, NC_), 1)
    corr = jnp.sum(jnp.where(cio == labv, delta, 0.0))
    lane = lax.broadcasted_iota(jnp.int32, (1, 128), 1)
    st_ref[0, 0] = (jnp.where(lane == 0, bce, 0.0) + jnp.where(lane == 1, num, 0.0)
                    + jnp.where(lane == 2, dsp, 0.0) + jnp.where(lane == 3, dst, 0.0)
                    + jnp.where(lane == 4, corr, 0.0))

    @pl.when(t == 0)
    def _():
        rk_ref[0] = jnp.full((1, HW), float(NP_), jnp.float32)

    fsi = si_ref[b, t].astype(jnp.float32)
    cur = rk_ref[0]
    rk_ref[0] = jnp.where(pos > 0.5, jnp.minimum(cur, fsi), cur)


def _k5(si, ti, labs, pm, tm, plg):
    grid_spec = pltpu.PrefetchScalarGridSpec(
        num_scalar_prefetch=3,
        grid=(N, T),
        in_specs=[
            pl.BlockSpec((1, 1, 1, HW), lambda b, t, si_r, ti_r, lb_r: (b, si_r[b, t], 0, 0)),
            pl.BlockSpec((1, 1, 1, HW), lambda b, t, si_r, ti_r, lb_r: (b, ti_r[b, t], 0, 0)),
            pl.BlockSpec((1, 1, 1, NC_), lambda b, t, si_r, ti_r, lb_r: (b, si_r[b, t], 0, 0)),
        ],
        out_specs=[
            pl.BlockSpec((1, 1, 1, 128), lambda b, t, si_r, ti_r, lb_r: (b, t, 0, 0)),
            pl.BlockSpec((1, 1, HW), lambda b, t, si_r, ti_r, lb_r: (b, 0, 0)),
        ],
    )
    return pl.pallas_call(
        _post_body,
        grid_spec=grid_spec,
        out_shape=[
            jax.ShapeDtypeStruct((N, T, 1, 128), jnp.float32),
            jax.ShapeDtypeStruct((N, 1, HW), jnp.float32),
        ],
    )(si, ti, labs, pm.reshape(N, NP_, 1, HW), tm.reshape(N, T, 1, HW),
      plg.reshape(N, NP_, 1, NC_))


RT_TILE = 2048
NRT = HW // RT_TILE
NB = 104


def _rank_body(pm_ref, rk_ref, ht_ref, hc_ref):
    b = pl.program_id(0)
    j = pl.program_id(1)
    x = pm_ref[0]
    m = jnp.max(x, axis=0, keepdims=True)
    lse = m + jnp.log(jnp.sum(jnp.exp(x - m), axis=0, keepdims=True))
    rank = rk_ref[0, 0:1, :].astype(jnp.int32)
    ridx = jnp.minimum(rank, NP_ - 1)
    rio = lax.broadcasted_iota(jnp.int32, (NP_, RT_TILE), 0)
    picked = jnp.sum(jnp.where(rio == ridx, x, 0.0), axis=0, keepdims=True)
    term = lse - picked
    bio = lax.broadcasted_iota(jnp.int32, (NB, RT_TILE), 0)
    oh = jnp.where(bio == rank, 1.0, 0.0)
    dn_l = (((1,), (1,)), ((), ()))
    tcon = lax.dot_general(oh, term, dn_l, preferred_element_type=jnp.float32)
    ccon = jnp.sum(oh, axis=1, keepdims=True)
    tconb = jnp.broadcast_to(tcon, (NB, 128))
    cconb = jnp.broadcast_to(ccon, (NB, 128))

    @pl.when((b == 0) & (j == 0))
    def _():
        ht_ref[...] = tconb
        hc_ref[...] = cconb

    @pl.when((b > 0) | (j > 0))
    def _():
        ht_ref[...] = ht_ref[...] + tconb
        hc_ref[...] = hc_ref[...] + cconb


def _k6(pm, rk):
    return pl.pallas_call(
        _rank_body,
        grid=(N, NRT),
        in_specs=[
            pl.BlockSpec((1, NP_, RT_TILE), lambda b, j: (b, 0, j)),
            pl.BlockSpec((1, 1, RT_TILE), lambda b, j: (b, 0, j)),
        ],
        out_specs=[
            pl.BlockSpec((NB, 128), lambda b, j: (0, 0)),
            pl.BlockSpec((NB, 128), lambda b, j: (0, 0)),
        ],
        out_shape=[
            jax.ShapeDtypeStruct((NB, 128), jnp.float32),
            jax.ShapeDtypeStruct((NB, 128), jnp.float32),
        ],
    )(pm, rk)


@jax.jit
def kernel(seg_mask_pred, sem_seg, feature_map, pred_masks, pred_logits, tgt_masks, tgt_labels):
    pm = pred_masks.reshape(N, NP_, HW)
    tm = tgt_masks.reshape(N, T, HW)
    lab3 = tgt_labels.reshape(N, 1, T)

    gum = _gumbel_const() if _GUMBEL is None else jnp.asarray(_GUMBEL)
    misc, keys, meta = _k1(pm, tm, pred_logits, lab3, gum.reshape(N, 1, HW))
    si_f = misc[:, 0, 0:T]
    ti_f = misc[:, 0, T:2 * T]
    term0 = jnp.sum(misc[:, 0, 2 * T])
    si = si_f.astype(jnp.int32)
    ti = ti_f.astype(jnp.int32)

    seg = _k2(seg_mask_pred.reshape(N, NC_, HW), sem_seg.reshape(N, NSEG, 1, SEG_TILE))
    ce_sum = jnp.sum(seg[:, 0, 0])
    npos = jnp.maximum(jnp.sum(seg[:, 0, 1]), 1.0)
    loss_seg = SEG_W * ce_sum / npos

    pos4 = _k1b(keys.reshape(N, 128, 128), meta)
    fmT = feature_map.reshape(N, CF, HW).transpose(0, 2, 1)
    tmT = tm.transpose(0, 2, 1)
    packed = jnp.concatenate(
        [fmT, tmT, jnp.zeros((N, HW, 128 - CF - T), jnp.float32)], axis=-1)
    sel = _sc_select(pos4, packed)

    inst = _k34(sel)
    loss_inst = INST_W * (jnp.sum(inst[:, 0, 0]) - jnp.sum(inst[:, 0, 1])) / (N * KS)

    stats4, rk = _k5(si, ti, tgt_labels, pm, tm, pred_logits)
    stats = stats4[:, :, 0, :]
    bce_sum = jnp.sum(stats[:, :, 0])
    loss_mask = MASK_W * bce_sum / (N * T * HW)
    numr = 2.0 * stats[:, :, 1]
    denr = stats[:, :, 2] + stats[:, :, 3]
    loss_dice = DICE_W * jnp.mean(1.0 - (numr + 1.0) / (denr + 1.0))
    corr = jnp.sum(stats[:, :, 4])
    loss_cls = CLS_W * (term0 + corr) / float(N * T)

    ht, hc = _k6(pm, rk)
    htc = ht[:, 0]
    hcc = hc[:, 0]
    ign = jnp.max(jnp.where(hcc > 0.0, jnp.arange(NB), -1))
    loss_rank = RANK_W * (jnp.sum(htc) - htc[ign]) / float(N * HW)

    return jnp.stack([loss_seg, loss_inst, loss_cls, loss_mask, loss_dice, loss_rank])


# speedup vs baseline: 1.0050x; 1.0050x over previous
import functools
import jax
import jax.numpy as jnp
import numpy as np
from jax import lax
from jax.experimental import pallas as pl
from jax.experimental.pallas import tpu as pltpu
from jax.experimental.pallas import tpu_sc as plsc

N = 2; T = 16; H = 128; W = 128; CF = 64
NP_ = 100; NC_ = 133; IGNORE = 255
RANK_W = 0.1; SEG_W = 1.0; MASK_W = 1.0; DICE_W = 4.0; CLS_W = 2.0; INST_W = 1.0
KS = 4096; ST = 0.3; MC = -99999.0
HW = H * W
JT = 512
NJ = KS // JT

def _gumbel_const():
    return -jnp.log(-jnp.log(jax.random.uniform(
        jax.random.key(42), (N, HW), minval=1e-6, maxval=1.0 - 1e-6)))

try:
    _GUMBEL = np.asarray(_gumbel_const())
except Exception:
    _GUMBEL = None


def _softplus(x):
    return jnp.maximum(x, 0.0) + jnp.log1p(jnp.exp(-jnp.abs(x)))


def _match_body(pm_ref, tm_ref, plg_ref, lab_ref, gum_ref, misc_ref, key_ref, meta_ref):
    x = pm_ref[0]
    t = tm_ref[0]
    p = jnp.clip(jax.nn.sigmoid(x), 1e-6, 1.0 - 1e-6)
    dn = (((1,), (1,)), ((), ()))
    pt = lax.dot_general(p, t, dn, preferred_element_type=jnp.float32)
    ones_hw = jnp.ones((1, HW), jnp.float32)
    t_area = lax.dot_general(ones_hw, t, dn, preferred_element_type=jnp.float32)
    p_sum = jnp.sum(p, axis=1, keepdims=True)
    mask_cost = (t_area + p_sum - 2.0 * pt) / HW
    dice_cost = -(2.0 * pt) / (p_sum + t_area + 1e-6)
    xl = plg_ref[0]
    prob = jax.nn.sigmoid(xl)
    neg = 0.75 * prob * prob * (-jnp.log(1.0 - prob + 1e-8))
    pos = 0.25 * (1.0 - prob) * (1.0 - prob) * (-jnp.log(prob + 1e-8))
    pn = pos - neg
    lab = lab_ref[0]
    ciota = lax.broadcasted_iota(jnp.int32, (NC_, T), 0)
    oh = jnp.where(ciota == lab, 1.0, 0.0)
    clsc = lax.dot_general(pn, oh, (((1,), (0,)), ((), ())),
                           preferred_element_type=jnp.float32)
    cost0 = MASK_W * mask_cost + DICE_W * dice_cost + CLS_W * clsc

    riota = lax.broadcasted_iota(jnp.int32, (NP_, T), 0)
    cio = lax.broadcasted_iota(jnp.int32, (NP_, T), 1)
    flat = riota * T + cio
    lane = lax.broadcasted_iota(jnp.int32, (1, 128), 1)

    def step(s, carry):
        cost, misc = carry
        mn = jnp.min(cost)
        fi = jnp.min(jnp.where(cost == mn, flat, 10 ** 9))
        i = fi // T
        j = fi - i * T
        cost = jnp.where((riota == i) | (cio == j), jnp.inf, cost)
        misc = jnp.where(lane == s, i.astype(jnp.float32), misc)
        misc = jnp.where(lane == T + s, j.astype(jnp.float32), misc)
        return cost, misc

    _, misc = lax.fori_loop(0, T, step, (cost0, jnp.zeros((1, 128), jnp.float32)))

    term0 = jnp.sum(0.75 * prob * prob * _softplus(xl))
    misc = misc + jnp.where(lane == 2 * T, term0, 0.0)

    tmr = jnp.round(t)
    area = lax.dot_general(ones_hw, tmr, dn, preferred_element_type=jnp.float32)
    dn2 = (((1,), (0,)), ((), ()))
    pix = lax.dot_general(area, tmr, dn2, preferred_element_type=jnp.float32)
    pix = jnp.where(pix == 0.0, 1.0, pix)
    cover = lax.dot_general(jnp.ones((1, T), jnp.float32), tmr, dn2,
                            preferred_element_type=jnp.float32)
    nonvoid = jnp.where(cover == 0.0, 0.0, 1.0)
    z = jnp.log(HW / pix) * ST + (1.0 - nonvoid) * MC + gum_ref[0]
    bits = lax.bitcast_convert_type(z, jnp.int32)
    keys = bits ^ lax.shift_right_arithmetic(bits, 31).astype(jnp.int32).__and__(
        jnp.int32(0x7FFFFFFF))
    key_ref[0] = keys
    cnt0 = jnp.sum(jnp.where(keys >= 0, 1.0, 0.0))
    lo0 = jnp.where(cnt0 >= KS, jnp.int32(0), jnp.int32(-2147483648))
    hi0 = jnp.where(cnt0 >= KS, jnp.int32(2147483647), jnp.int32(-1))

    def bstep(_, carry):
        lo, hi = carry
        d = hi - lo
        mid = lo + lax.shift_right_logical(d, 1) + (d & 1)
        cnt = jnp.sum(jnp.where(keys >= mid, 1.0, 0.0))
        ok = cnt >= KS
        return jnp.where(ok, mid, lo), jnp.where(ok, hi, mid - 1)

    thr, _ = lax.fori_loop(0, 31, bstep, (lo0, hi0))
    g = jnp.sum(jnp.where(keys > thr, 1.0, 0.0)).astype(jnp.int32)
    needed = KS - g
    r3 = lax.broadcasted_iota(jnp.int32, (3, 16), 0)
    meta_ref[0] = jnp.where(r3 == 0, thr, jnp.where(r3 == 1, g, needed))
    misc_ref[0] = misc


def _k1(pm, tm, plg, lab, gum):
    return pl.pallas_call(
        _match_body,
        grid=(N,),
        in_specs=[
            pl.BlockSpec((1, NP_, HW), lambda b: (b, 0, 0)),
            pl.BlockSpec((1, T, HW), lambda b: (b, 0, 0)),
            pl.BlockSpec((1, NP_, NC_), lambda b: (b, 0, 0)),
            pl.BlockSpec((1, 1, T), lambda b: (b, 0, 0)),
            pl.BlockSpec((1, 1, HW), lambda b: (b, 0, 0)),
        ],
        out_specs=[
            pl.BlockSpec((1, 1, 128), lambda b: (b, 0, 0)),
            pl.BlockSpec((1, 1, HW), lambda b: (b, 0, 0)),
            pl.BlockSpec((1, 3, 16), lambda b: (b, 0, 0)),
        ],
        out_shape=[
            jax.ShapeDtypeStruct((N, 1, 128), jnp.float32),
            jax.ShapeDtypeStruct((N, 1, HW), jnp.int32),
            jax.ShapeDtypeStruct((N, 3, 16), jnp.int32),
        ],
    )(pm, tm, plg, lab, gum)


CHUNK = HW // 16
NV = CHUNK // 16
OUT_LEN = KS + HW


def _pos_body(key_ref, meta_ref, pos_ref):
    x = key_ref[0]
    mrow = meta_ref[0]
    thr = jnp.max(mrow[0:1, :])
    g = jnp.max(mrow[1:2, :]).astype(jnp.float32)
    needed = jnp.max(mrow[2:3, :]).astype(jnp.float32)
    gt = x > thr
    eq = x == thr
    ri = lax.broadcasted_iota(jnp.int32, (128, 128), 0)
    ci = lax.broadcasted_iota(jnp.int32, (128, 128), 1)
    ut = jnp.where(ri <= ci, 1.0, 0.0)
    ls = jnp.where(ci < ri, 1.0, 0.0)
    dn = (((1,), (0,)), ((), ()))
    gtf = jnp.where(gt, 1.0, 0.0)
    eqf = jnp.where(eq, 1.0, 0.0)
    gp = lax.dot_general(gtf, ut, dn, preferred_element_type=jnp.float32)
    go = lax.dot_general(ls, gp[:, 127:128], dn, preferred_element_type=jnp.float32)
    gpos = gp + go - 1.0
    ep = lax.dot_general(eqf, ut, dn, preferred_element_type=jnp.float32)
    eo = lax.dot_general(ls, ep[:, 127:128], dn, preferred_element_type=jnp.float32)
    epos = ep + eo - 1.0
    sel_eq = eq & (epos < needed)
    pixf = (ri * 128 + ci).astype(jnp.float32)
    posf = jnp.where(gt, gpos, jnp.where(sel_eq, g + epos, float(KS) + pixf))
    pos_ref[0] = posf.astype(jnp.int32).reshape(16, 8, 128)


def _k1b(keys3, meta):
    return pl.pallas_call(
        _pos_body,
        grid=(N,),
        in_specs=[
            pl.BlockSpec((1, 128, 128), lambda b: (b, 0, 0)),
            pl.BlockSpec((1, 3, 16), lambda b: (b, 0, 0)),
        ],
        out_specs=pl.BlockSpec((1, 16, 8, 128), lambda b: (b, 0, 0, 0)),
        out_shape=jax.ShapeDtypeStruct((N, 16, 8, 128), jnp.int32),
    )(keys3, meta)


def _sc_sel_body(pos_hbm, packed_hbm, out_hbm, posb, buf0, buf1, sem):
    b = lax.axis_index("c")
    chunk = lax.axis_index("s")
    base = chunk * CHUNK
    pltpu.sync_copy(pos_hbm.at[b, chunk], posb)
    bufs = [buf0, buf1]
    descs = []
    for s in range(8):
        buf = bufs[s % 2]
        if s >= 2:
            descs[s - 2].wait()
        pltpu.sync_copy(packed_hbm.at[b, pl.ds(base + s * 128, 128)], buf)
        descs.append(pltpu.async_copy(buf, out_hbm.at[b].at[posb.at[s]], sem))
    descs[6].wait()
    descs[7].wait()


def _sc_select(pos4, packed):
    mesh = plsc.VectorSubcoreMesh(core_axis_name="c", subcore_axis_name="s")
    fn = functools.partial(
        pl.kernel,
        mesh=mesh,
        out_type=jax.ShapeDtypeStruct((N, OUT_LEN, 128), jnp.float32),
        scratch_types=[
            pltpu.VMEM((8, 128), jnp.int32),
            pltpu.VMEM((128, 128), jnp.float32),
            pltpu.VMEM((128, 128), jnp.float32),
            pltpu.SemaphoreType.DMA,
        ],
    )(_sc_sel_body)
    return fn(pos4, packed)


SEG_TILE = 4096
NSEG = HW // SEG_TILE


def _seg_body(sp_ref, ss_ref, out_ref):
    j = pl.program_id(1)
    x = sp_ref[0]
    m = jnp.max(x, axis=0, keepdims=True)
    lse = m + jnp.log(jnp.sum(jnp.exp(x - m), axis=0, keepdims=True))
    idx = ss_ref[0, 0]
    valid = (idx >= 0) & (idx < NC_) & (idx != IGNORE)
    vf = valid.astype(jnp.float32)
    idxc = jnp.clip(idx, 0, NC_ - 1)
    rio = lax.broadcasted_iota(jnp.int32, (NC_, SEG_TILE), 0)
    xg = jnp.sum(jnp.where(rio == idxc, x, 0.0), axis=0, keepdims=True)
    s1 = jnp.sum((lse - xg) * vf)
    s2 = jnp.sum(vf)
    lane = lax.broadcasted_iota(jnp.int32, (1, 128), 1)
    contrib = jnp.where(lane == 0, s1, 0.0) + jnp.where(lane == 1, s2, 0.0)

    @pl.when(j == 0)
    def _():
        out_ref[0] = contrib

    @pl.when(j > 0)
    def _():
        out_ref[0] = out_ref[0] + contrib


def _k2(smp, ss3):
    return pl.pallas_call(
        _seg_body,
        grid=(N, NSEG),
        in_specs=[
            pl.BlockSpec((1, NC_, SEG_TILE), lambda b, j: (b, 0, j)),
            pl.BlockSpec((1, 1, 1, SEG_TILE), lambda b, j: (b, j, 0, 0)),
        ],
        out_specs=pl.BlockSpec((1, 1, 128), lambda b, j: (b, 0, 0)),
        out_shape=jax.ShapeDtypeStruct((N, 1, 128), jnp.float32),
    )(smp, ss3)


def _inst_body(sel_ref, out_ref):
    x = sel_ref[0]
    f = x[:, :CF]
    nrm = jnp.sqrt(jnp.sum(f * f, axis=1, keepdims=True))
    fn = f / jnp.maximum(nrm, 1e-12)
    a = jnp.round(x[:, CF:CF + T])
    cnt = jnp.sum(a, axis=0, keepdims=True)
    dn_l = (((1,), (1,)), ((), ()))
    nc = lax.dot_general(a, cnt, dn_l, preferred_element_type=jnp.float32)
    ncw = jnp.where(nc == 0.0, 1.0, nc)
    w = a / ncw
    wi = jnp.sum(w, axis=0, keepdims=True)
    v = lax.dot_general(w, fn, (((0,), (0,)), ((), ())),
                        preferred_element_type=jnp.float32)
    q = lax.dot_general(wi, a, dn_l, preferred_element_type=jnp.float32)
    av = lax.dot_general(a, v, (((1,), (0,)), ((), ())),
                         preferred_element_type=jnp.float32)
    r_sum = jnp.sum(av * fn) / ST

    ones_row = jnp.ones((1, JT), jnp.float32)
    cs = [jnp.zeros((1, JT), jnp.float32) for _ in range(NJ)]
    tiles = [fn[ti * JT:(ti + 1) * JT, :] for ti in range(NJ)]
    for ti in range(NJ):
        for tj in range(ti, NJ):
            s = lax.dot_general(tiles[ti], tiles[tj], dn_l,
                                preferred_element_type=jnp.float32) * (1.0 / ST)
            e = jnp.exp(s)
            cs[tj] = cs[tj] + lax.dot_general(ones_row, e, (((1,), (0,)), ((), ())),
                                              preferred_element_type=jnp.float32)
            if tj > ti:
                cs[ti] = cs[ti] + lax.dot_general(ones_row, e, dn_l,
                                                  preferred_element_type=jnp.float32)
    cq = jnp.zeros((), jnp.float32)
    for tj in range(NJ):
        qc = q[:, tj * JT:(tj + 1) * JT]
        cq = cq + jnp.sum(jnp.log(cs[tj]) * qc)
    lane = lax.broadcasted_iota(jnp.int32, (1, 128), 1)
    out_ref[0] = jnp.where(lane == 0, cq, 0.0) + jnp.where(lane == 1, r_sum, 0.0)


def _k34(sel):
    return pl.pallas_call(
        _inst_body,
        grid=(N,),
        in_specs=[
            pl.BlockSpec((1, KS, 128), lambda b: (b, 0, 0)),
        ],
        out_specs=pl.BlockSpec((1, 1, 128), lambda b: (b, 0, 0)),
        out_shape=jax.ShapeDtypeStruct((N, 1, 128), jnp.float32),
    )(sel)


def _post_body(si_ref, ti_ref, lab_ref, pm_ref, tm_ref, plg_ref, st_ref, rk_ref):
    b = pl.program_id(0)
    t = pl.program_id(1)
    x = pm_ref[0, 0]
    pos = tm_ref[0, 0]
    bce = jnp.sum(jnp.maximum(x, 0.0) - x * pos + jnp.log1p(jnp.exp(-jnp.abs(x))))
    sig = jax.nn.sigmoid(x)
    num = jnp.sum(sig * pos)
    dsp = jnp.sum(sig)
    dst = jnp.sum(pos)
    row = plg_ref[0, 0]
    p1 = jax.nn.sigmoid(row)
    delta = 0.25 * (1.0 - p1) * (1.0 - p1) * _softplus(-row) \
        - 0.75 * p1 * p1 * _softplus(row)
    ti = ti_ref[b, t]
    labv = lab_ref[b, ti]
    cio = lax.broadcasted_iota(jnp.int32, (1, NC_), 1)
    corr = jnp.sum(jnp.where(cio == labv, delta, 0.0))
    lane = lax.broadcasted_iota(jnp.int32, (1, 128), 1)
    st_ref[0, 0] = (jnp.where(lane == 0, bce, 0.0) + jnp.where(lane == 1, num, 0.0)
                    + jnp.where(lane == 2, dsp, 0.0) + jnp.where(lane == 3, dst, 0.0)
                    + jnp.where(lane == 4, corr, 0.0))

    @pl.when(t == 0)
    def _():
        rk_ref[0] = jnp.full((1, HW), float(NP_), jnp.float32)

    fsi = si_ref[b, t].astype(jnp.float32)
    cur = rk_ref[0]
    rk_ref[0] = jnp.where(pos > 0.5, jnp.minimum(cur, fsi), cur)


def _k5(si, ti, labs, pm, tm, plg):
    grid_spec = pltpu.PrefetchScalarGridSpec(
        num_scalar_prefetch=3,
        grid=(N, T),
        in_specs=[
            pl.BlockSpec((1, 1, 1, HW), lambda b, t, si_r, ti_r, lb_r: (b, si_r[b, t], 0, 0)),
            pl.BlockSpec((1, 1, 1, HW), lambda b, t, si_r, ti_r, lb_r: (b, ti_r[b, t], 0, 0)),
            pl.BlockSpec((1, 1, 1, NC_), lambda b, t, si_r, ti_r, lb_r: (b, si_r[b, t], 0, 0)),
        ],
        out_specs=[
            pl.BlockSpec((1, 1, 1, 128), lambda b, t, si_r, ti_r, lb_r: (b, t, 0, 0)),
            pl.BlockSpec((1, 1, HW), lambda b, t, si_r, ti_r, lb_r: (b, 0, 0)),
        ],
    )
    return pl.pallas_call(
        _post_body,
        grid_spec=grid_spec,
        out_shape=[
            jax.ShapeDtypeStruct((N, T, 1, 128), jnp.float32),
            jax.ShapeDtypeStruct((N, 1, HW), jnp.float32),
        ],
    )(si, ti, labs, pm.reshape(N, NP_, 1, HW), tm.reshape(N, T, 1, HW),
      plg.reshape(N, NP_, 1, NC_))


RT_TILE = 2048
NRT = HW // RT_TILE
NB = 104


def _rank_body(pm_ref, rk_ref, ht_ref, hc_ref):
    b = pl.program_id(0)
    j = pl.program_id(1)
    x = pm_ref[0]
    m = jnp.max(x, axis=0, keepdims=True)
    lse = m + jnp.log(jnp.sum(jnp.exp(x - m), axis=0, keepdims=True))
    rank = rk_ref[0, 0:1, :].astype(jnp.int32)
    ridx = jnp.minimum(rank, NP_ - 1)
    rio = lax.broadcasted_iota(jnp.int32, (NP_, RT_TILE), 0)
    picked = jnp.sum(jnp.where(rio == ridx, x, 0.0), axis=0, keepdims=True)
    term = lse - picked
    bio = lax.broadcasted_iota(jnp.int32, (NB, RT_TILE), 0)
    oh = jnp.where(bio == rank, 1.0, 0.0)
    dn_l = (((1,), (1,)), ((), ()))
    tcon = lax.dot_general(oh, term, dn_l, preferred_element_type=jnp.float32)
    ccon = jnp.sum(oh, axis=1, keepdims=True)
    tconb = jnp.broadcast_to(tcon, (NB, 128))
    cconb = jnp.broadcast_to(ccon, (NB, 128))

    @pl.when((b == 0) & (j == 0))
    def _():
        ht_ref[...] = tconb
        hc_ref[...] = cconb

    @pl.when((b > 0) | (j > 0))
    def _():
        ht_ref[...] = ht_ref[...] + tconb
        hc_ref[...] = hc_ref[...] + cconb


def _k6(pm, rk):
    return pl.pallas_call(
        _rank_body,
        grid=(N, NRT),
        in_specs=[
            pl.BlockSpec((1, NP_, RT_TILE), lambda b, j: (b, 0, j)),
            pl.BlockSpec((1, 1, RT_TILE), lambda b, j: (b, 0, j)),
        ],
        out_specs=[
            pl.BlockSpec((NB, 128), lambda b, j: (0, 0)),
            pl.BlockSpec((NB, 128), lambda b, j: (0, 0)),
        ],
        out_shape=[
            jax.ShapeDtypeStruct((NB, 128), jnp.float32),
            jax.ShapeDtypeStruct((NB, 128), jnp.float32),
        ],
    )(pm, rk)


@jax.jit
def kernel(seg_mask_pred, sem_seg, feature_map, pred_masks, pred_logits, tgt_masks, tgt_labels):
    pm = pred_masks.reshape(N, NP_, HW)
    tm = tgt_masks.reshape(N, T, HW)
    lab3 = tgt_labels.reshape(N, 1, T)

    gum = _gumbel_const() if _GUMBEL is None else jnp.asarray(_GUMBEL)
    misc, keys, meta = _k1(pm, tm, pred_logits, lab3, gum.reshape(N, 1, HW))
    si_f = misc[:, 0, 0:T]
    ti_f = misc[:, 0, T:2 * T]
    term0 = jnp.sum(misc[:, 0, 2 * T])
    si = si_f.astype(jnp.int32)
    ti = ti_f.astype(jnp.int32)

    seg = _k2(seg_mask_pred.reshape(N, NC_, HW), sem_seg.reshape(N, NSEG, 1, SEG_TILE))
    ce_sum = jnp.sum(seg[:, 0, 0])
    npos = jnp.maximum(jnp.sum(seg[:, 0, 1]), 1.0)
    loss_seg = SEG_W * ce_sum / npos

    pos4 = _k1b(keys.reshape(N, 128, 128), meta)
    fmT = feature_map.reshape(N, CF, HW).transpose(0, 2, 1)
    tmT = tm.transpose(0, 2, 1)
    packed = jnp.concatenate(
        [fmT, tmT, jnp.zeros((N, HW, 128 - CF - T), jnp.float32)], axis=-1)
    sel = _sc_select(pos4, packed)

    inst = _k34(sel)
    loss_inst = INST_W * (jnp.sum(inst[:, 0, 0]) - jnp.sum(inst[:, 0, 1])) / (N * KS)

    stats4, rk = _k5(si, ti, tgt_labels, pm, tm, pred_logits)
    stats = stats4[:, :, 0, :]
    bce_sum = jnp.sum(stats[:, :, 0])
    loss_mask = MASK_W * bce_sum / (N * T * HW)
    numr = 2.0 * stats[:, :, 1]
    denr = stats[:, :, 2] + stats[:, :, 3]
    loss_dice = DICE_W * jnp.mean(1.0 - (numr + 1.0) / (denr + 1.0))
    corr = jnp.sum(stats[:, :, 4])
    loss_cls = CLS_W * (term0 + corr) / float(N * T)

    ht, hc = _k6(pm, rk)
    htc = ht[:, 0]
    hcc = hc[:, 0]
    ign = jnp.max(jnp.where(hcc > 0.0, jnp.arange(NB), -1))
    loss_rank = RANK_W * (jnp.sum(htc) - htc[ign]) / float(N * HW)

    return jnp.stack([loss_seg, loss_inst, loss_cls, loss_mask, loss_dice, loss_rank])


# speedup vs baseline: 1.0380x; 1.0328x over previous
import functools
import jax
import jax.numpy as jnp
import numpy as np
from jax import lax
from jax.experimental import pallas as pl
from jax.experimental.pallas import tpu as pltpu
from jax.experimental.pallas import tpu_sc as plsc

N = 2; T = 16; H = 128; W = 128; CF = 64
NP_ = 100; NC_ = 133; IGNORE = 255
RANK_W = 0.1; SEG_W = 1.0; MASK_W = 1.0; DICE_W = 4.0; CLS_W = 2.0; INST_W = 1.0
KS = 4096; ST = 0.3; MC = -99999.0
HW = H * W
JT = 512
NJ = KS // JT

def _gumbel_const():
    return -jnp.log(-jnp.log(jax.random.uniform(
        jax.random.key(42), (N, HW), minval=1e-6, maxval=1.0 - 1e-6)))

try:
    _GUMBEL = np.asarray(_gumbel_const())
except Exception:
    _GUMBEL = None


def _softplus(x):
    return jnp.maximum(x, 0.0) + jnp.log1p(jnp.exp(-jnp.abs(x)))


def _match_body(pm_ref, tm_ref, plg_ref, lab_ref, gum_ref, misc_ref, key_ref, meta_ref):
    x = pm_ref[0]
    t = tm_ref[0]
    p = jnp.clip(jax.nn.sigmoid(x), 1e-6, 1.0 - 1e-6)
    dn = (((1,), (1,)), ((), ()))
    pt = lax.dot_general(p, t, dn, preferred_element_type=jnp.float32)
    ones_hw = jnp.ones((1, HW), jnp.float32)
    t_area = lax.dot_general(ones_hw, t, dn, preferred_element_type=jnp.float32)
    p_sum = jnp.sum(p, axis=1, keepdims=True)
    mask_cost = (t_area + p_sum - 2.0 * pt) / HW
    dice_cost = -(2.0 * pt) / (p_sum + t_area + 1e-6)
    xl = plg_ref[0]
    prob = jax.nn.sigmoid(xl)
    neg = 0.75 * prob * prob * (-jnp.log(1.0 - prob + 1e-8))
    pos = 0.25 * (1.0 - prob) * (1.0 - prob) * (-jnp.log(prob + 1e-8))
    pn = pos - neg
    lab = lab_ref[0]
    ciota = lax.broadcasted_iota(jnp.int32, (NC_, T), 0)
    oh = jnp.where(ciota == lab, 1.0, 0.0)
    clsc = lax.dot_general(pn, oh, (((1,), (0,)), ((), ())),
                           preferred_element_type=jnp.float32)
    cost0 = MASK_W * mask_cost + DICE_W * dice_cost + CLS_W * clsc

    riota = lax.broadcasted_iota(jnp.int32, (NP_, T), 0)
    cio = lax.broadcasted_iota(jnp.int32, (NP_, T), 1)
    flat = riota * T + cio
    lane = lax.broadcasted_iota(jnp.int32, (1, 128), 1)

    def step(s, carry):
        cost, misc = carry
        mn = jnp.min(cost)
        fi = jnp.min(jnp.where(cost == mn, flat, 10 ** 9))
        i = fi // T
        j = fi - i * T
        cost = jnp.where((riota == i) | (cio == j), jnp.inf, cost)
        misc = jnp.where(lane == s, i.astype(jnp.float32), misc)
        misc = jnp.where(lane == T + s, j.astype(jnp.float32), misc)
        return cost, misc

    _, misc = lax.fori_loop(0, T, step, (cost0, jnp.zeros((1, 128), jnp.float32)))

    term0 = jnp.sum(0.75 * prob * prob * _softplus(xl))
    misc = misc + jnp.where(lane == 2 * T, term0, 0.0)

    tmr = jnp.round(t)
    area = lax.dot_general(ones_hw, tmr, dn, preferred_element_type=jnp.float32)
    dn2 = (((1,), (0,)), ((), ()))
    pix = lax.dot_general(area, tmr, dn2, preferred_element_type=jnp.float32)
    pix = jnp.where(pix == 0.0, 1.0, pix)
    cover = lax.dot_general(jnp.ones((1, T), jnp.float32), tmr, dn2,
                            preferred_element_type=jnp.float32)
    nonvoid = jnp.where(cover == 0.0, 0.0, 1.0)
    z = jnp.log(HW / pix) * ST + (1.0 - nonvoid) * MC + gum_ref[0]
    bits = lax.bitcast_convert_type(z, jnp.int32)
    keys = bits ^ lax.shift_right_arithmetic(bits, 31).astype(jnp.int32).__and__(
        jnp.int32(0x7FFFFFFF))
    key_ref[0] = keys
    cnt0 = jnp.sum(jnp.where(keys >= 0, 1.0, 0.0))
    lo0 = jnp.where(cnt0 >= KS, jnp.int32(0), jnp.int32(-2147483648))
    hi0 = jnp.where(cnt0 >= KS, jnp.int32(2147483647), jnp.int32(-1))

    def bstep(_, carry):
        lo, hi = carry
        d = hi - lo
        mid = lo + lax.shift_right_logical(d, 1) + (d & 1)
        cnt = jnp.sum(jnp.where(keys >= mid, 1.0, 0.0))
        ok = cnt >= KS
        return jnp.where(ok, mid, lo), jnp.where(ok, hi, mid - 1)

    thr, _ = lax.fori_loop(0, 31, bstep, (lo0, hi0))
    g = jnp.sum(jnp.where(keys > thr, 1.0, 0.0)).astype(jnp.int32)
    needed = KS - g
    r3 = lax.broadcasted_iota(jnp.int32, (3, 16), 0)
    meta_ref[0] = jnp.where(r3 == 0, thr, jnp.where(r3 == 1, g, needed))
    misc_ref[0] = misc


def _k1(pm, tm, plg, lab, gum):
    return pl.pallas_call(
        _match_body,
        grid=(N,),
        in_specs=[
            pl.BlockSpec((1, NP_, HW), lambda b: (b, 0, 0)),
            pl.BlockSpec((1, T, HW), lambda b: (b, 0, 0)),
            pl.BlockSpec((1, NP_, NC_), lambda b: (b, 0, 0)),
            pl.BlockSpec((1, 1, T), lambda b: (b, 0, 0)),
            pl.BlockSpec((1, 1, HW), lambda b: (b, 0, 0)),
        ],
        out_specs=[
            pl.BlockSpec((1, 1, 128), lambda b: (b, 0, 0)),
            pl.BlockSpec((1, 1, HW), lambda b: (b, 0, 0)),
            pl.BlockSpec((1, 3, 16), lambda b: (b, 0, 0)),
        ],
        out_shape=[
            jax.ShapeDtypeStruct((N, 1, 128), jnp.float32),
            jax.ShapeDtypeStruct((N, 1, HW), jnp.int32),
            jax.ShapeDtypeStruct((N, 3, 16), jnp.int32),
        ],
    )(pm, tm, plg, lab, gum)


CHUNK = HW // 16
NV = CHUNK // 16
OUT_LEN = KS + HW


def _pos_body(key_ref, meta_ref, pos_ref):
    x = key_ref[0]
    mrow = meta_ref[0]
    thr = jnp.max(mrow[0:1, :])
    g = jnp.max(mrow[1:2, :]).astype(jnp.float32)
    needed = jnp.max(mrow[2:3, :]).astype(jnp.float32)
    gt = x > thr
    eq = x == thr
    ri = lax.broadcasted_iota(jnp.int32, (128, 128), 0)
    ci = lax.broadcasted_iota(jnp.int32, (128, 128), 1)
    ut = jnp.where(ri <= ci, 1.0, 0.0)
    ls = jnp.where(ci < ri, 1.0, 0.0)
    dn = (((1,), (0,)), ((), ()))
    gtf = jnp.where(gt, 1.0, 0.0)
    eqf = jnp.where(eq, 1.0, 0.0)
    gp = lax.dot_general(gtf, ut, dn, preferred_element_type=jnp.float32)
    go = lax.dot_general(ls, gp[:, 127:128], dn, preferred_element_type=jnp.float32)
    gpos = gp + go - 1.0
    ep = lax.dot_general(eqf, ut, dn, preferred_element_type=jnp.float32)
    eo = lax.dot_general(ls, ep[:, 127:128], dn, preferred_element_type=jnp.float32)
    epos = ep + eo - 1.0
    sel_eq = eq & (epos < needed)
    pixf = (ri * 128 + ci).astype(jnp.float32)
    posf = jnp.where(gt, gpos, jnp.where(sel_eq, g + epos, float(KS) + pixf))
    pos_ref[0] = posf.astype(jnp.int32).reshape(16, 8, 128)


def _k1b(keys3, meta):
    return pl.pallas_call(
        _pos_body,
        grid=(N,),
        in_specs=[
            pl.BlockSpec((1, 128, 128), lambda b: (b, 0, 0)),
            pl.BlockSpec((1, 3, 16), lambda b: (b, 0, 0)),
        ],
        out_specs=pl.BlockSpec((1, 16, 8, 128), lambda b: (b, 0, 0, 0)),
        out_shape=jax.ShapeDtypeStruct((N, 16, 8, 128), jnp.int32),
    )(keys3, meta)


def _sc_sel_body(pos_hbm, packed_hbm, out_hbm, posb, buf0, buf1, sem):
    b = lax.axis_index("c")
    chunk = lax.axis_index("s")
    base = chunk * CHUNK
    pltpu.sync_copy(pos_hbm.at[b, chunk], posb)
    bufs = [buf0, buf1]
    descs = []
    for s in range(8):
        buf = bufs[s % 2]
        if s >= 2:
            descs[s - 2].wait()
        pltpu.sync_copy(packed_hbm.at[b, pl.ds(base + s * 128, 128)], buf)
        descs.append(pltpu.async_copy(buf, out_hbm.at[b].at[posb.at[s]], sem))
    descs[6].wait()
    descs[7].wait()


def _sc_select(pos4, packed):
    mesh = plsc.VectorSubcoreMesh(core_axis_name="c", subcore_axis_name="s")
    fn = functools.partial(
        pl.kernel,
        mesh=mesh,
        out_type=jax.ShapeDtypeStruct((N, OUT_LEN, 128), jnp.float32),
        scratch_types=[
            pltpu.VMEM((8, 128), jnp.int32),
            pltpu.VMEM((128, 128), jnp.float32),
            pltpu.VMEM((128, 128), jnp.float32),
            pltpu.SemaphoreType.DMA,
        ],
    )(_sc_sel_body)
    return fn(pos4, packed)


SEG_TILE = 4096
NSEG = HW // SEG_TILE


def _seg_body(sp_ref, ss_ref, out_ref):
    j = pl.program_id(1)
    x = sp_ref[0]
    m = jnp.max(x, axis=0, keepdims=True)
    lse = m + jnp.log(jnp.sum(jnp.exp(x - m), axis=0, keepdims=True))
    idx = ss_ref[0, 0]
    valid = (idx >= 0) & (idx < NC_) & (idx != IGNORE)
    vf = valid.astype(jnp.float32)
    idxc = jnp.clip(idx, 0, NC_ - 1)
    rio = lax.broadcasted_iota(jnp.int32, (NC_, SEG_TILE), 0)
    xg = jnp.sum(jnp.where(rio == idxc, x, 0.0), axis=0, keepdims=True)
    s1 = jnp.sum((lse - xg) * vf)
    s2 = jnp.sum(vf)
    lane = lax.broadcasted_iota(jnp.int32, (1, 128), 1)
    contrib = jnp.where(lane == 0, s1, 0.0) + jnp.where(lane == 1, s2, 0.0)

    @pl.when(j == 0)
    def _():
        out_ref[0] = contrib

    @pl.when(j > 0)
    def _():
        out_ref[0] = out_ref[0] + contrib


def _k2(smp, ss3):
    return pl.pallas_call(
        _seg_body,
        grid=(N, NSEG),
        in_specs=[
            pl.BlockSpec((1, NC_, SEG_TILE), lambda b, j: (b, 0, j)),
            pl.BlockSpec((1, 1, 1, SEG_TILE), lambda b, j: (b, j, 0, 0)),
        ],
        out_specs=pl.BlockSpec((1, 1, 128), lambda b, j: (b, 0, 0)),
        out_shape=jax.ShapeDtypeStruct((N, 1, 128), jnp.float32),
    )(smp, ss3)


def _inst_body(sel_ref, out_ref):
    x = sel_ref[0]
    f = x[:, :CF]
    nrm = jnp.sqrt(jnp.sum(f * f, axis=1, keepdims=True))
    fn = f / jnp.maximum(nrm, 1e-12)
    a = jnp.round(x[:, CF:CF + T])
    cnt = jnp.sum(a, axis=0, keepdims=True)
    dn_l = (((1,), (1,)), ((), ()))
    nc = lax.dot_general(a, cnt, dn_l, preferred_element_type=jnp.float32)
    ncw = jnp.where(nc == 0.0, 1.0, nc)
    w = a / ncw
    wi = jnp.sum(w, axis=0, keepdims=True)
    v = lax.dot_general(w, fn, (((0,), (0,)), ((), ())),
                        preferred_element_type=jnp.float32)
    q = lax.dot_general(wi, a, dn_l, preferred_element_type=jnp.float32)
    av = lax.dot_general(a, v, (((1,), (0,)), ((), ())),
                         preferred_element_type=jnp.float32)
    r_sum = jnp.sum(av * fn) / ST

    ones_row = jnp.ones((1, JT), jnp.float32)
    cs = [jnp.zeros((1, JT), jnp.float32) for _ in range(NJ)]
    tiles = [fn[ti * JT:(ti + 1) * JT, :] for ti in range(NJ)]
    for ti in range(NJ):
        for tj in range(ti, NJ):
            s = lax.dot_general(tiles[ti], tiles[tj], dn_l,
                                preferred_element_type=jnp.float32) * (1.0 / ST)
            e = jnp.exp(s)
            cs[tj] = cs[tj] + lax.dot_general(ones_row, e, (((1,), (0,)), ((), ())),
                                              preferred_element_type=jnp.float32)
            if tj > ti:
                cs[ti] = cs[ti] + lax.dot_general(ones_row, e, dn_l,
                                                  preferred_element_type=jnp.float32)
    cq = jnp.zeros((), jnp.float32)
    for tj in range(NJ):
        qc = q[:, tj * JT:(tj + 1) * JT]
        cq = cq + jnp.sum(jnp.log(cs[tj]) * qc)
    lane = lax.broadcasted_iota(jnp.int32, (1, 128), 1)
    out_ref[0] = jnp.where(lane == 0, cq, 0.0) + jnp.where(lane == 1, r_sum, 0.0)


def _k34(sel):
    return pl.pallas_call(
        _inst_body,
        grid=(N,),
        in_specs=[
            pl.BlockSpec((1, KS, 128), lambda b: (b, 0, 0)),
        ],
        out_specs=pl.BlockSpec((1, 1, 128), lambda b: (b, 0, 0)),
        out_shape=jax.ShapeDtypeStruct((N, 1, 128), jnp.float32),
    )(sel)


def _post_body(si_ref, ti_ref, lab_ref, pm0_ref, pm1_ref, tm0_ref, tm1_ref,
               plg0_ref, plg1_ref, st_ref, rk_ref):
    b = pl.program_id(0)
    t = pl.program_id(1)
    lane = lax.broadcasted_iota(jnp.int32, (1, 128), 1)
    cio = lax.broadcasted_iota(jnp.int32, (1, NC_), 1)
    outv = jnp.zeros((1, 128), jnp.float32)

    @pl.when(t == 0)
    def _():
        rk_ref[0] = jnp.full((1, HW), float(NP_), jnp.float32)

    cur = rk_ref[0]
    for h, (pm_r, tm_r, plg_r) in enumerate(
            [(pm0_ref, tm0_ref, plg0_ref), (pm1_ref, tm1_ref, plg1_ref)]):
        x = pm_r[0, 0]
        pos = tm_r[0, 0]
        bce = jnp.sum(jnp.maximum(x, 0.0) - x * pos + jnp.log1p(jnp.exp(-jnp.abs(x))))
        sig = jax.nn.sigmoid(x)
        num = jnp.sum(sig * pos)
        dsp = jnp.sum(sig)
        dst = jnp.sum(pos)
        row = plg_r[0, 0]
        p1 = jax.nn.sigmoid(row)
        delta = 0.25 * (1.0 - p1) * (1.0 - p1) * _softplus(-row) \
            - 0.75 * p1 * p1 * _softplus(row)
        ti = ti_ref[b, 2 * t + h]
        labv = lab_ref[b, ti]
        corr = jnp.sum(jnp.where(cio == labv, delta, 0.0))
        o = 8 * h
        outv = (outv + jnp.where(lane == o, bce, 0.0) + jnp.where(lane == o + 1, num, 0.0)
                + jnp.where(lane == o + 2, dsp, 0.0) + jnp.where(lane == o + 3, dst, 0.0)
                + jnp.where(lane == o + 4, corr, 0.0))
        fsi = si_ref[b, 2 * t + h].astype(jnp.float32)
        cur = jnp.where(pos > 0.5, jnp.minimum(cur, fsi), cur)
    st_ref[0, 0] = outv
    rk_ref[0] = cur


def _k5(si, ti, labs, pm, tm, plg):
    grid_spec = pltpu.PrefetchScalarGridSpec(
        num_scalar_prefetch=3,
        grid=(N, T // 2),
        in_specs=[
            pl.BlockSpec((1, 1, 1, HW), lambda b, t, si_r, ti_r, lb_r: (b, si_r[b, 2 * t], 0, 0)),
            pl.BlockSpec((1, 1, 1, HW), lambda b, t, si_r, ti_r, lb_r: (b, si_r[b, 2 * t + 1], 0, 0)),
            pl.BlockSpec((1, 1, 1, HW), lambda b, t, si_r, ti_r, lb_r: (b, ti_r[b, 2 * t], 0, 0)),
            pl.BlockSpec((1, 1, 1, HW), lambda b, t, si_r, ti_r, lb_r: (b, ti_r[b, 2 * t + 1], 0, 0)),
            pl.BlockSpec((1, 1, 1, NC_), lambda b, t, si_r, ti_r, lb_r: (b, si_r[b, 2 * t], 0, 0)),
            pl.BlockSpec((1, 1, 1, NC_), lambda b, t, si_r, ti_r, lb_r: (b, si_r[b, 2 * t + 1], 0, 0)),
        ],
        out_specs=[
            pl.BlockSpec((1, 1, 1, 128), lambda b, t, si_r, ti_r, lb_r: (b, t, 0, 0)),
            pl.BlockSpec((1, 1, HW), lambda b, t, si_r, ti_r, lb_r: (b, 0, 0)),
        ],
    )
    return pl.pallas_call(
        _post_body,
        grid_spec=grid_spec,
        out_shape=[
            jax.ShapeDtypeStruct((N, T // 2, 1, 128), jnp.float32),
            jax.ShapeDtypeStruct((N, 1, HW), jnp.float32),
        ],
    )(si, ti, labs, pm.reshape(N, NP_, 1, HW), pm.reshape(N, NP_, 1, HW),
      tm.reshape(N, T, 1, HW), tm.reshape(N, T, 1, HW),
      plg.reshape(N, NP_, 1, NC_), plg.reshape(N, NP_, 1, NC_))


RT_TILE = 2048
NRT = HW // RT_TILE
NB = 104


def _rank_body(pm_ref, rk_ref, ht_ref, hc_ref):
    b = pl.program_id(0)
    j = pl.program_id(1)
    x = pm_ref[0]
    m = jnp.max(x, axis=0, keepdims=True)
    lse = m + jnp.log(jnp.sum(jnp.exp(x - m), axis=0, keepdims=True))
    rank = rk_ref[0, 0:1, :].astype(jnp.int32)
    ridx = jnp.minimum(rank, NP_ - 1)
    rio = lax.broadcasted_iota(jnp.int32, (NP_, RT_TILE), 0)
    picked = jnp.sum(jnp.where(rio == ridx, x, 0.0), axis=0, keepdims=True)
    term = lse - picked
    bio = lax.broadcasted_iota(jnp.int32, (NB, RT_TILE), 0)
    oh = jnp.where(bio == rank, 1.0, 0.0)
    dn_l = (((1,), (1,)), ((), ()))
    tcon = lax.dot_general(oh, term, dn_l, preferred_element_type=jnp.float32)
    ccon = jnp.sum(oh, axis=1, keepdims=True)
    tconb = jnp.broadcast_to(tcon, (NB, 128))
    cconb = jnp.broadcast_to(ccon, (NB, 128))

    @pl.when((b == 0) & (j == 0))
    def _():
        ht_ref[...] = tconb
        hc_ref[...] = cconb

    @pl.when((b > 0) | (j > 0))
    def _():
        ht_ref[...] = ht_ref[...] + tconb
        hc_ref[...] = hc_ref[...] + cconb


def _k6(pm, rk):
    return pl.pallas_call(
        _rank_body,
        grid=(N, NRT),
        in_specs=[
            pl.BlockSpec((1, NP_, RT_TILE), lambda b, j: (b, 0, j)),
            pl.BlockSpec((1, 1, RT_TILE), lambda b, j: (b, 0, j)),
        ],
        out_specs=[
            pl.BlockSpec((NB, 128), lambda b, j: (0, 0)),
            pl.BlockSpec((NB, 128), lambda b, j: (0, 0)),
        ],
        out_shape=[
            jax.ShapeDtypeStruct((NB, 128), jnp.float32),
            jax.ShapeDtypeStruct((NB, 128), jnp.float32),
        ],
    )(pm, rk)


@jax.jit
def kernel(seg_mask_pred, sem_seg, feature_map, pred_masks, pred_logits, tgt_masks, tgt_labels):
    pm = pred_masks.reshape(N, NP_, HW)
    tm = tgt_masks.reshape(N, T, HW)
    lab3 = tgt_labels.reshape(N, 1, T)

    gum = _gumbel_const() if _GUMBEL is None else jnp.asarray(_GUMBEL)
    misc, keys, meta = _k1(pm, tm, pred_logits, lab3, gum.reshape(N, 1, HW))
    si_f = misc[:, 0, 0:T]
    ti_f = misc[:, 0, T:2 * T]
    term0 = jnp.sum(misc[:, 0, 2 * T])
    si = si_f.astype(jnp.int32)
    ti = ti_f.astype(jnp.int32)

    seg = _k2(seg_mask_pred.reshape(N, NC_, HW), sem_seg.reshape(N, NSEG, 1, SEG_TILE))
    ce_sum = jnp.sum(seg[:, 0, 0])
    npos = jnp.maximum(jnp.sum(seg[:, 0, 1]), 1.0)
    loss_seg = SEG_W * ce_sum / npos

    pos4 = _k1b(keys.reshape(N, 128, 128), meta)
    fmT = feature_map.reshape(N, CF, HW).transpose(0, 2, 1)
    tmT = tm.transpose(0, 2, 1)
    packed = jnp.concatenate(
        [fmT, tmT, jnp.zeros((N, HW, 128 - CF - T), jnp.float32)], axis=-1)
    sel = _sc_select(pos4, packed)

    inst = _k34(sel)
    loss_inst = INST_W * (jnp.sum(inst[:, 0, 0]) - jnp.sum(inst[:, 0, 1])) / (N * KS)

    stats4, rk = _k5(si, ti, tgt_labels, pm, tm, pred_logits)
    stats = stats4[:, :, 0, :]
    bce_sum = jnp.sum(stats[:, :, 0] + stats[:, :, 8])
    loss_mask = MASK_W * bce_sum / (N * T * HW)
    d0 = 1.0 - (2.0 * stats[:, :, 1] + 1.0) / (stats[:, :, 2] + stats[:, :, 3] + 1.0)
    d1 = 1.0 - (2.0 * stats[:, :, 9] + 1.0) / (stats[:, :, 10] + stats[:, :, 11] + 1.0)
    loss_dice = DICE_W * jnp.mean(jnp.concatenate([d0, d1], axis=1))
    corr = jnp.sum(stats[:, :, 4] + stats[:, :, 12])
    loss_cls = CLS_W * (term0 + corr) / float(N * T)

    ht, hc = _k6(pm, rk)
    htc = ht[:, 0]
    hcc = hc[:, 0]
    ign = jnp.max(jnp.where(hcc > 0.0, jnp.arange(NB), -1))
    loss_rank = RANK_W * (jnp.sum(htc) - htc[ign]) / float(N * HW)

    return jnp.stack([loss_seg, loss_inst, loss_cls, loss_mask, loss_dice, loss_rank])
